# baseline (speedup 1.0000x reference)
import functools

import numpy as np
import jax
import jax.numpy as jnp
from jax import lax
from jax.experimental import pallas as pl
from jax.experimental.pallas import tpu as pltpu
from jax.experimental.pallas import tpu_sc as plsc

_NUM_FIELDS = 26
_FIELD_DIM = 100000
_EMBED_DIM = 32
_LANES = 16
_PERIOD = 208

_NC = 2
_NS = 16
_NW = _NC * _NS

_CHUNK = 1664
_IDX_ROW = 128

_OFFPAT = ((np.arange(_PERIOD) % _NUM_FIELDS) * _FIELD_DIM).astype(np.int32)


def _sc_gather(x_flat, table, offpat, n_total):
    per_w = n_total // _NW
    n_chunks = per_w // _CHUNK
    rows_per_chunk = _CHUNK // _IDX_ROW

    mesh = plsc.VectorSubcoreMesh(core_axis_name="c", subcore_axis_name="s")

    @functools.partial(
        pl.kernel,
        mesh=mesh,
        out_type=jax.ShapeDtypeStruct((n_total, _EMBED_DIM), jnp.float32),
        scratch_types=[
            pltpu.VMEM((_CHUNK,), jnp.int32),
            pltpu.VMEM((_CHUNK, _EMBED_DIM), jnp.float32),
            pltpu.VMEM((_PERIOD,), jnp.int32),
            pltpu.SemaphoreType.DMA,
        ],
        compiler_params=pltpu.CompilerParams(use_tc_tiling_on_sc=False),
    )
    def body(x_hbm, table_hbm, off_hbm, out_hbm, idx_v, rows_v, offp_v, sem):
        wid = lax.axis_index("s") * _NC + lax.axis_index("c")
        base = wid * per_w
        pltpu.sync_copy(off_hbm, offp_v)

        def chunk_body(c, carry):
            pltpu.sync_copy(x_hbm.at[pl.ds(base + c * _CHUNK, _CHUNK)], idx_v)
            for i in range(_CHUNK // _LANES):
                s = (i * _LANES) % _PERIOD
                idx_v[pl.ds(i * _LANES, _LANES)] = (
                    idx_v[pl.ds(i * _LANES, _LANES)] + offp_v[pl.ds(s, _LANES)]
                )
            copies = [
                pltpu.async_copy(
                    table_hbm.at[idx_v.at[pl.ds(j * _IDX_ROW, _IDX_ROW)]],
                    rows_v.at[pl.ds(j * _IDX_ROW, _IDX_ROW)],
                    sem,
                )
                for j in range(rows_per_chunk)
            ]
            for cp in copies:
                cp.wait()
            pltpu.sync_copy(rows_v, out_hbm.at[pl.ds(base + c * _CHUNK, _CHUNK)])
            return carry

        lax.fori_loop(0, n_chunks, chunk_body, 0)

    return body(x_flat, table, offpat)


def kernel(x, table):
    b, f = x.shape
    n_total = b * f
    x_flat = x.reshape(n_total)
    offpat = jnp.asarray(_OFFPAT)
    out = _sc_gather(x_flat, table, offpat, n_total)
    return out.reshape(b, f, _EMBED_DIM)
